# baseline (speedup 1.0000x reference)
import functools

import jax
import jax.numpy as jnp
from jax import lax
from jax.experimental import pallas as pl
from jax.experimental.pallas import tpu as pltpu
from jax.experimental.pallas import tpu_sc as plsc

NUM_CORES = 2
NUM_SUBCORES = 16
NUM_WORKERS = NUM_CORES * NUM_SUBCORES
CROWS = 4
OUTW = 128


def _embedding_lookup(idx2d, table):
    batch, seq = idx2d.shape
    dim = table.shape[1]
    rows_per_w = batch // NUM_WORKERS
    n_chunks = rows_per_w // CROWS
    n_pairs = n_chunks // 2
    mesh = plsc.VectorSubcoreMesh(core_axis_name="c", subcore_axis_name="s")

    @functools.partial(
        pl.kernel,
        mesh=mesh,
        out_type=jax.ShapeDtypeStruct((batch, seq, OUTW), jnp.float32),
        scratch_types=[
            pltpu.VMEM((rows_per_w, seq), jnp.int32),
            pltpu.VMEM((CROWS, seq, dim), jnp.float32),
            pltpu.VMEM((CROWS, seq, dim), jnp.float32),
            pltpu.SemaphoreType.DMA,
            pltpu.SemaphoreType.DMA,
            pltpu.SemaphoreType.DMA,
            pltpu.SemaphoreType.DMA,
        ],
        compiler_params=pltpu.CompilerParams(use_tc_tiling_on_sc=False),
    )
    def emb(idx_hbm, table_hbm, out_hbm, idx_v, buf0, buf1,
            gsem0, gsem1, wsem0, wsem1):
        wid = lax.axis_index("s") * NUM_CORES + lax.axis_index("c")
        base = wid * rows_per_w
        pltpu.sync_copy(idx_hbm.at[pl.ds(base, rows_per_w)], idx_v)

        def fire_gather(chunk_row, buf, sem):
            for j in range(CROWS):
                pltpu.async_copy(
                    table_hbm.at[idx_v.at[chunk_row + j]],
                    buf.at[j],
                    sem,
                )

        def drain_gather(buf, sem):
            for j in range(CROWS):
                pltpu.make_async_copy(
                    table_hbm.at[idx_v.at[j]],
                    buf.at[j],
                    sem,
                ).wait()

        def fire_write(buf, chunk_row, sem):
            pltpu.async_copy(
                buf,
                out_hbm.at[pl.ds(base + chunk_row, CROWS), :, pl.ds(0, dim)],
                sem,
            )

        def drain_write(buf, sem):
            pltpu.make_async_copy(
                buf,
                out_hbm.at[pl.ds(base, CROWS), :, pl.ds(0, dim)],
                sem,
            ).wait()

        fire_gather(0, buf0, gsem0)

        def pair_body(t, carry):
            c1_row = (2 * t + 1) * CROWS
            c2_row = (2 * t + 2) * CROWS

            @pl.when(t > 0)
            def _():
                drain_write(buf1, wsem1)

            fire_gather(c1_row, buf1, gsem1)
            drain_gather(buf0, gsem0)
            fire_write(buf0, 2 * t * CROWS, wsem0)
            drain_write(buf0, wsem0)

            @pl.when(2 * t + 2 < n_chunks)
            def _():
                fire_gather(c2_row, buf0, gsem0)

            drain_gather(buf1, gsem1)
            fire_write(buf1, c1_row, wsem1)
            return carry

        lax.fori_loop(0, n_pairs, pair_body, 0)
        drain_write(buf1, wsem1)

    return emb(idx2d, table)


def kernel(input, table):
    dim = table.shape[1]
    out_full = _embedding_lookup(input, table)
    return out_full[:, :, :dim]


# speedup vs baseline: 1.0747x; 1.0747x over previous
import functools

import jax
import jax.numpy as jnp
from jax import lax
from jax.experimental import pallas as pl
from jax.experimental.pallas import tpu as pltpu
from jax.experimental.pallas import tpu_sc as plsc

NUM_CORES = 2
NUM_SUBCORES = 16
NUM_WORKERS = NUM_CORES * NUM_SUBCORES
CHUNK = 512
GATHER = 256
N_G = CHUNK // GATHER


def _embedding_lookup(idx_flat, table):
    total = idx_flat.shape[0]
    dim = table.shape[1]
    b_per_w = total // NUM_WORKERS
    n_chunks = b_per_w // CHUNK
    n_pairs = n_chunks // 2
    mesh = plsc.VectorSubcoreMesh(core_axis_name="c", subcore_axis_name="s")

    @functools.partial(
        pl.kernel,
        mesh=mesh,
        out_type=jax.ShapeDtypeStruct((total, dim), jnp.float32),
        scratch_types=[
            pltpu.VMEM((b_per_w,), jnp.int32),
            pltpu.VMEM((CHUNK, dim), jnp.float32),
            pltpu.VMEM((CHUNK, dim), jnp.float32),
            pltpu.SemaphoreType.DMA,
            pltpu.SemaphoreType.DMA,
            pltpu.SemaphoreType.DMA,
            pltpu.SemaphoreType.DMA,
        ],
        compiler_params=pltpu.CompilerParams(use_tc_tiling_on_sc=False),
    )
    def emb(idx_hbm, table_hbm, out_hbm, idx_v, buf0, buf1,
            gsem0, gsem1, wsem0, wsem1):
        wid = lax.axis_index("s") * NUM_CORES + lax.axis_index("c")
        base = wid * b_per_w
        pltpu.sync_copy(idx_hbm.at[pl.ds(base, b_per_w)], idx_v)

        def fire_gathers(chunk_row, buf, sem):
            for j in range(N_G):
                pltpu.async_copy(
                    table_hbm.at[idx_v.at[pl.ds(chunk_row + j * GATHER, GATHER)]],
                    buf.at[pl.ds(j * GATHER, GATHER)],
                    sem,
                )

        def drain_gathers(buf, sem):
            for j in range(N_G):
                pltpu.make_async_copy(
                    table_hbm.at[idx_v.at[pl.ds(j * GATHER, GATHER)]],
                    buf.at[pl.ds(j * GATHER, GATHER)],
                    sem,
                ).wait()

        def fire_write(buf, chunk_row, sem):
            pltpu.async_copy(buf, out_hbm.at[pl.ds(base + chunk_row, CHUNK)], sem)

        def drain_write(buf, sem):
            pltpu.make_async_copy(
                buf, out_hbm.at[pl.ds(base, CHUNK)], sem
            ).wait()

        fire_gathers(0, buf0, gsem0)

        def pair_body(t, carry):
            c1_row = (2 * t + 1) * CHUNK
            c2_row = (2 * t + 2) * CHUNK

            @pl.when(t > 0)
            def _():
                drain_write(buf1, wsem1)

            fire_gathers(c1_row, buf1, gsem1)
            drain_gathers(buf0, gsem0)
            fire_write(buf0, 2 * t * CHUNK, wsem0)
            drain_write(buf0, wsem0)

            @pl.when(2 * t + 2 < n_chunks)
            def _():
                fire_gathers(c2_row, buf0, gsem0)

            drain_gathers(buf1, gsem1)
            fire_write(buf1, c1_row, wsem1)
            return carry

        lax.fori_loop(0, n_pairs, pair_body, 0)
        drain_write(buf1, wsem1)

    return emb(idx_flat, table)


def kernel(input, table):
    B, L = input.shape
    dim = table.shape[1]
    idx_flat = input.reshape(B * L)
    out = _embedding_lookup(idx_flat, table)
    return out.reshape(B, L, dim)
